# baseline (speedup 1.0000x reference)
import jax
import jax.numpy as jnp
from jax.experimental import pallas as pl
from jax.experimental.pallas import tpu as pltpu

_TEMPERATURE = 0.25
_THRESHOLD = 0.5
_KC = 2
_VC = 4


def _dot_nt(a, b):
    return jax.lax.dot_general(a, b, (((1,), (1,)), ((), ())),
                               preferred_element_type=jnp.float32)


def _dot_nn(a, b):
    return jax.lax.dot_general(a, b, (((1,), (0,)), ((), ())),
                               preferred_element_type=jnp.float32)


def _body(q_ref, wq_ref, wr_ref, keys_hbm, vals_hbm, out_ref,
          keys_v, vals_v, ksem, vsem):
    B, Q, QD = q_ref.shape
    EDp1, M = keys_hbm.shape
    kc = M // _KC
    mc = M // _VC

    kcopies = [
        pltpu.make_async_copy(keys_hbm.at[:, pl.ds(h * kc, kc)],
                              keys_v.at[:, pl.ds(h * kc, kc)],
                              ksem.at[h])
        for h in range(_KC)
    ]
    for cp in kcopies:
        cp.start()

    def vcopy(b, c):
        return pltpu.make_async_copy(
            vals_hbm.at[b, :, pl.ds(c * mc, mc)],
            vals_v.at[b % 2, :, pl.ds(c * mc, mc)],
            vsem.at[b % 2, c])

    for b in range(min(2, B)):
        for c in range(_VC):
            vcopy(b, c).start()

    def pow8(s):
        s2 = s * s
        s4 = s2 * s2
        return s4 * s4

    for b in range(B):
        q = q_ref[b]
        qe = _dot_nt(q, wq_ref[...])
        e = jnp.exp(qe * (1.0 / _TEMPERATURE))
        denom = 1.0 + jnp.sum(e, axis=-1, keepdims=True)
        num = jnp.concatenate([e, jnp.ones_like(denom)], axis=-1)
        qs = jnp.sqrt(num / denom)
        s8 = []
        for h in range(_KC):
            if b == 0:
                kcopies[h].wait()
            s8.append(pow8(_dot_nn(qs, keys_v[:, h * kc:(h + 1) * kc])))
        mx = s8[0].max(axis=-1, keepdims=True)
        for s in s8[1:]:
            mx = jnp.maximum(mx, s.max(axis=-1, keepdims=True))
        thr = jnp.where(mx < _THRESHOLD, 0.9 * mx, _THRESHOLD)
        masked = [jnp.where(s < thr, 0.0, s) for s in s8]
        wsum = masked[0].sum(axis=-1, keepdims=True)
        for ms in masked[1:]:
            wsum = wsum + ms.sum(axis=-1, keepdims=True)
        read = None
        for c in range(_VC):
            vcopy(b, c).wait()
            wslice = masked[(c * mc) // kc][:, (c * mc) % kc:(c * mc) % kc + mc]
            contrib = _dot_nt(wslice, vals_v[b % 2, :, c * mc:(c + 1) * mc])
            read = contrib if read is None else read + contrib
        if b + 2 < B:
            for c in range(_VC):
                vcopy(b + 2, c).start()
        read = read / (wsum + 1e-9)
        out_ref[b] = _dot_nn(read, wr_ref[...])


def kernel(queries, W_query, W_read, memory_keys, memory_values):
    B, Q, QD = queries.shape
    VD, ED = W_read.shape
    M, EDp1 = memory_keys.shape
    keys_t = memory_keys.T
    vals_t = memory_values.transpose(0, 2, 1)
    wr_t = W_read.T
    return pl.pallas_call(
        _body,
        in_specs=[
            pl.BlockSpec((B, Q, QD), lambda: (0, 0, 0)),
            pl.BlockSpec((ED, QD), lambda: (0, 0)),
            pl.BlockSpec((ED, VD), lambda: (0, 0)),
            pl.BlockSpec(memory_space=pl.ANY),
            pl.BlockSpec(memory_space=pl.ANY),
        ],
        out_specs=pl.BlockSpec((B, Q, VD), lambda: (0, 0, 0)),
        out_shape=jax.ShapeDtypeStruct((B, Q, VD), jnp.float32),
        scratch_shapes=[
            pltpu.VMEM((EDp1, M), jnp.float32),
            pltpu.VMEM((2, ED, M), jnp.float32),
            pltpu.SemaphoreType.DMA((_KC,)),
            pltpu.SemaphoreType.DMA((2, _VC)),
        ],
    )(queries, W_query, wr_t, keys_t, vals_t)


# speedup vs baseline: 1.0505x; 1.0505x over previous
import jax
import jax.numpy as jnp
from jax.experimental import pallas as pl
from jax.experimental.pallas import tpu as pltpu

_TEMPERATURE = 0.25
_THRESHOLD = 0.5


def _dot_nt(a, b):
    return jax.lax.dot_general(a, b, (((1,), (1,)), ((), ())),
                               preferred_element_type=jnp.float32)


def _dot_nn(a, b):
    return jax.lax.dot_general(a, b, (((1,), (0,)), ((), ())),
                               preferred_element_type=jnp.float32)


_SK = 2
_SV = 4


def _body(q_ref, wq_ref, wr_ref, *rest):
    keys_refs = rest[:_SK]
    vals_refs = rest[_SK:_SK + _SV]
    out_ref = rest[_SK + _SV]
    q = q_ref[0]
    qe = _dot_nt(q, wq_ref[...])
    e = jnp.exp(qe * (1.0 / _TEMPERATURE))
    denom = 1.0 + jnp.sum(e, axis=-1, keepdims=True)
    num = jnp.concatenate([e, jnp.ones_like(denom)], axis=-1)
    qs = jnp.sqrt(num / denom)
    scores = [_dot_nn(qs, k[...]) for k in keys_refs]

    def pow8(s):
        s2 = s * s
        s4 = s2 * s2
        return s4 * s4

    s8 = [pow8(s) for s in scores]
    mx = s8[0].max(axis=-1, keepdims=True)
    for s in s8[1:]:
        mx = jnp.maximum(mx, s.max(axis=-1, keepdims=True))
    thr = jnp.where(mx < _THRESHOLD, 0.9 * mx, _THRESHOLD)
    masked = [jnp.where(s < thr, 0.0, s) for s in s8]
    wsum = masked[0].sum(axis=-1, keepdims=True)
    for ms in masked[1:]:
        wsum = wsum + ms.sum(axis=-1, keepdims=True)
    mh = masked[0].shape[1]
    vh = vals_refs[0].shape[2]
    read = None
    for i, v in enumerate(vals_refs):
        wslice = masked[(i * vh) // mh][:, (i * vh) % mh:(i * vh) % mh + vh]
        c = _dot_nt(wslice, v[0])
        read = c if read is None else read + c
    read = read / (wsum + 1e-9)
    out_ref[0] = _dot_nn(read, wr_ref[...])


def kernel(queries, W_query, W_read, memory_keys, memory_values):
    B, Q, QD = queries.shape
    VD, ED = W_read.shape
    M, EDp1 = memory_keys.shape
    keys_t = memory_keys.T
    vals_t = memory_values.transpose(0, 2, 1)
    wr_t = W_read.T
    kh = M // _SK
    vh = M // _SV
    kspecs = [
        pl.BlockSpec((EDp1, kh), lambda b, i=i: (0, i)) for i in range(_SK)
    ]
    vspecs = [
        pl.BlockSpec((1, ED, vh), lambda b, i=i: (b, 0, i)) for i in range(_SV)
    ]
    return pl.pallas_call(
        _body,
        grid=(B,),
        in_specs=[
            pl.BlockSpec((1, Q, QD), lambda b: (b, 0, 0)),
            pl.BlockSpec((ED, QD), lambda b: (0, 0)),
            pl.BlockSpec((ED, VD), lambda b: (0, 0)),
            *kspecs,
            *vspecs,
        ],
        out_specs=pl.BlockSpec((1, Q, VD), lambda b: (b, 0, 0)),
        out_shape=jax.ShapeDtypeStruct((B, Q, VD), jnp.float32),
        compiler_params=pltpu.CompilerParams(
            dimension_semantics=("arbitrary",)),
    )(queries, W_query, wr_t, *([keys_t] * _SK), *([vals_t] * _SV))
